# baseline (speedup 1.0000x reference)
import functools
import math

import jax
import jax.numpy as jnp
from jax import lax
from jax.experimental import pallas as pl
from jax.experimental.pallas import tpu as pltpu

N_NODES, U_NUM, I_NUM = 10000, 3000, 7000


def _mm_body(x_ref, w_ref, o_ref, *, relu_in):
    x = x_ref[...]
    if relu_in:
        x = jnp.maximum(x, 0.0)
    o_ref[...] = jnp.dot(x, w_ref[...], preferred_element_type=jnp.float32)


def _mm_scale_body(x_ref, s_ref, w_ref, o_ref, *, relu_in):
    x = x_ref[...]
    if relu_in:
        x = jnp.maximum(x, 0.0)
    x = x * s_ref[...]
    o_ref[...] = jnp.dot(x, w_ref[...], preferred_element_type=jnp.float32)


def _mm(x, w, scale=None, relu_in=False, bm=512):
    n, d = x.shape
    npad = math.ceil(n / bm) * bm
    if npad != n:
        x = jnp.pad(x, ((0, npad - n), (0, 0)))
        if scale is not None:
            scale = jnp.pad(scale, ((0, npad - n), (0, 0)))
    grid = (npad // bm,)
    out_shape = jax.ShapeDtypeStruct((npad, w.shape[1]), jnp.float32)
    if scale is None:
        fn = pl.pallas_call(
            functools.partial(_mm_body, relu_in=relu_in),
            grid=grid,
            in_specs=[
                pl.BlockSpec((bm, d), lambda i: (i, 0)),
                pl.BlockSpec((d, w.shape[1]), lambda i: (0, 0)),
            ],
            out_specs=pl.BlockSpec((bm, w.shape[1]), lambda i: (i, 0)),
            out_shape=out_shape,
        )
        y = fn(x, w)
    else:
        fn = pl.pallas_call(
            functools.partial(_mm_scale_body, relu_in=relu_in),
            grid=grid,
            in_specs=[
                pl.BlockSpec((bm, d), lambda i: (i, 0)),
                pl.BlockSpec((bm, 1), lambda i: (i, 0)),
                pl.BlockSpec((d, w.shape[1]), lambda i: (0, 0)),
            ],
            out_specs=pl.BlockSpec((bm, w.shape[1]), lambda i: (i, 0)),
            out_shape=out_shape,
        )
        y = fn(x, scale, w)
    return y[:n]


def _mmoe_body(x_ref, ew_ref, eb_ref, gw_ref, tw1_ref, tb1_ref, tw2_ref,
               tb2_ref, o_ref):
    x = x_ref[...]
    eo = []
    for k in range(6):
        pre = jnp.dot(x, ew_ref[k], preferred_element_type=jnp.float32)
        eo.append(jnp.maximum(pre + eb_ref[k][None, :], 0.0))
    outs = []
    for t in range(2):
        logits = jnp.dot(x, gw_ref[t], preferred_element_type=jnp.float32)
        logits = logits - jnp.max(logits, axis=-1, keepdims=True)
        eg = jnp.exp(logits)
        g = eg / jnp.sum(eg, axis=-1, keepdims=True)
        mixed = g[:, 0:1] * eo[0]
        for k in range(1, 6):
            mixed = mixed + g[:, k:k + 1] * eo[k]
        t1 = jnp.dot(mixed, tw1_ref[t], preferred_element_type=jnp.float32)
        t1 = jnp.maximum(t1 + tb1_ref[t][None, :], 0.0)
        t2 = jnp.dot(t1, tw2_ref[t], preferred_element_type=jnp.float32)
        outs.append(t2[:, 0] + tb2_ref[t][0])
    o_ref[...] = jnp.stack(outs, axis=-1)


def _mmoe(x, expert_W, expert_b, gate_W, tower_W1, tower_b1, tower_W2,
          tower_b2, bm=1280):
    m, es = x.shape
    assert m % bm == 0
    grid = (m // bm,)
    full = lambda shp: pl.BlockSpec(shp, lambda i: tuple(0 for _ in shp))
    fn = pl.pallas_call(
        _mmoe_body,
        grid=grid,
        in_specs=[
            pl.BlockSpec((bm, es), lambda i: (i, 0)),
            full(expert_W.shape), full(expert_b.shape), full(gate_W.shape),
            full(tower_W1.shape), full(tower_b1.shape), full(tower_W2.shape),
            full(tower_b2.shape),
        ],
        out_specs=pl.BlockSpec((bm, 2), lambda i: (i, 0)),
        out_shape=jax.ShapeDtypeStruct((m, 2), jnp.float32),
    )
    return fn(x, expert_W, expert_b, gate_W, tower_W1, tower_b1, tower_W2,
              tower_b2)


def _seg_sum(x, src, dst, n):
    return jax.ops.segment_sum(x[src], dst, num_segments=n)


def _deg_inv(dst, n):
    deg = jax.ops.segment_sum(jnp.ones(dst.shape, jnp.float32), dst,
                              num_segments=n)
    return 1.0 / jnp.clip(deg, 1.0)


def kernel(target_user, item_sample, user_sample, embed, embed_ui, embed_pi,
           embed_u, W_hg, W_g1, W_g2, hyper_src, hyper_dst, ii_src, ii_dst,
           pi_src, pi_dst, ip_src, ip_dst, expert_W, expert_b, gate_W,
           tower_W1, tower_b1, tower_W2, tower_b2):
    N, U, I = N_NODES, U_NUM, I_NUM

    dinv_hyper = _deg_inv(hyper_dst, N)[:, None]
    dinv_ii = _deg_inv(ii_dst, N)[:, None]
    dinv_pi = _deg_inv(pi_dst, N)[:, None]
    dinv_ip = _deg_inv(ip_dst, U)[:, None]

    t0 = _mm(embed, W_hg)
    s_h = _seg_sum(t0, hyper_src, hyper_dst, N)

    def gcn(x_pre_sum, x_dinv, src, dst, dinv, n, relu_in):
        t1 = _mm(x_pre_sum, W_g1, scale=x_dinv, relu_in=relu_in)
        s1 = _seg_sum(t1, src, dst, n)
        t2 = _mm(s1, W_g2, scale=dinv, relu_in=True)
        s2 = _seg_sum(t2, src, dst, n)
        return s2 * dinv

    ones_n = jnp.ones((N, 1), jnp.float32)
    ones_u = jnp.ones((U, 1), jnp.float32)
    init_item_h = gcn(s_h, dinv_hyper, ii_src, ii_dst, dinv_ii, N, True)
    part_item_h = gcn(s_h, dinv_hyper, pi_src, pi_dst, dinv_pi, N, True)
    init_part_h = gcn(s_h[:U], dinv_hyper[:U], ip_src, ip_dst, dinv_ip, U,
                      True)
    init_item_g = gcn(embed_ui, ones_n, ii_src, ii_dst, dinv_ii, N, False)
    part_item_g = gcn(embed_pi, ones_n, pi_src, pi_dst, dinv_pi, N, False)
    init_part_g = gcn(embed_u, ones_u, ip_src, ip_dst, dinv_ip, U, False)

    init_item_embed = jnp.concatenate((init_item_h, init_item_g), axis=1)
    part_item_embed = jnp.concatenate((part_item_h, part_item_g), axis=1)
    init_part_embed = jnp.concatenate((init_part_h, init_part_g), axis=1)

    init_item_type = init_item_embed[:U]
    init_part_type = init_part_embed[:U]
    part_item_type = part_item_embed[:U]
    part_init_type = init_part_embed[:U]
    item_init_type = init_item_embed[U:U + I]
    item_part_type = part_item_embed[U:U + I]

    allp = jnp.mean(jnp.concatenate((part_item_type, part_init_type), axis=1),
                    axis=0, keepdims=True)
    tu = jnp.concatenate((init_item_type[target_user][:, None, :],
                          init_part_type[target_user][:, None, :]), axis=2)
    B, Si = item_sample.shape
    isf = item_sample.reshape(-1)
    item_sample_embed = jnp.concatenate(
        (item_init_type[isf].reshape(B, Si, -1),
         item_part_type[isf].reshape(B, Si, -1)), axis=2)
    Sp = user_sample.shape[1]
    usf = user_sample.reshape(-1)
    user_sample_embed = jnp.concatenate(
        (part_item_type[usf].reshape(B, Sp, -1),
         part_init_type[usf].reshape(B, Sp, -1)), axis=2)
    true_item = item_sample_embed[:, 0:1]
    users1 = jnp.tile(tu, (1, Si, 1))
    users2 = jnp.tile(tu, (1, Sp, 1))
    true_is = jnp.tile(true_item, (1, Sp, 1))
    allp_b = jnp.tile(allp[None], (B, Si, 1))
    u_isample_p = jnp.concatenate((users1, item_sample_embed, allp_b), axis=2)
    u_i_psample = jnp.concatenate((users2, true_is, user_sample_embed),
                                  axis=2)
    u_i_p = jnp.concatenate((u_isample_p, u_i_psample), axis=1)
    bs, ss, es = u_i_p.shape
    x = u_i_p.reshape(bs * ss, es)

    o = _mmoe(x, expert_W, expert_b, gate_W, tower_W1, tower_b1, tower_W2,
              tower_b2)
    output1 = o[:, 0].reshape(bs, ss)
    output2 = o[:, 1].reshape(bs, ss)
    loc = ss // 2
    task1_score = output1[:, :loc]
    task2_score = output2[:, loc:]

    def bpr(inp):
        return jnp.mean(-jax.nn.log_sigmoid(inp[:, 0:1] - inp[:, 1:]),
                        axis=-1)

    bprloss = 0.2 * bpr(task1_score[:, 0:5]) + bpr(task2_score[:, 0:5])
    truelabels = jnp.ones((bs, ss), jnp.float32).at[:, 1:loc].set(0.0)
    listloss = -jnp.sum(jax.nn.softmax(truelabels, axis=1)
                        * jnp.log(jax.nn.softmax(output1, axis=1)), axis=1)
    loss = bprloss + 0.3 * listloss + bpr(output2[:, :loc])
    return loss, task1_score, task2_score


# speedup vs baseline: 2.2635x; 2.2635x over previous
import functools
import math

import jax
import jax.numpy as jnp
from jax import lax
from jax.experimental import pallas as pl
from jax.experimental.pallas import tpu as pltpu
from jax.experimental.pallas import tpu_sc as plsc

N_NODES, U_NUM, I_NUM = 3000 + 7000, 3000, 7000
_NS, _NC, _L = 16, 2, 128


_G = 8


def _sc_segsum_body(x_ref, src_ref, dst_ref, z_ref, out_ref,
                    xs, outs, rows, stage, src_v, dst_v, gsem,
                    *, n_tab, n_groups):
    c = lax.axis_index("c")
    s = lax.axis_index("s")
    r_t = n_tab // _NS
    row0 = s * r_t
    n_slabs = r_t // _L
    pltpu.sync_copy(z_ref, rows)
    for v in range(n_slabs):
        pltpu.sync_copy(x_ref.at[c, pl.ds(row0 + v * _L, _L)], stage)
        pltpu.sync_copy(stage, xs.at[pl.ds(row0 + v * _L, _L)])
        pltpu.sync_copy(rows, outs.at[pl.ds(row0 + v * _L, _L)])
    plsc.subcore_barrier()

    @pl.loop(0, n_groups)
    def _grp(g):
        pltpu.sync_copy(src_ref.at[s, g], src_v)
        pltpu.sync_copy(dst_ref.at[s, g], dst_v)
        for b in range(_G):
            pltpu.async_copy(xs.at[src_v.at[b]], rows, gsem).wait()
            pltpu.sync_copy(rows, outs.at[dst_v.at[b]], add=True)

    plsc.subcore_barrier()
    for v in range(n_slabs):
        pltpu.sync_copy(outs.at[pl.ds(row0 + v * _L, _L)], stage)
        pltpu.sync_copy(stage, out_ref.at[c, pl.ds(row0 + v * _L, _L)])


def _sc_segsum(x, src, dst, n):
    n_tab = math.ceil((n + 8) / (_NS * _L)) * (_NS * _L)
    E = src.shape[0]
    epad = math.ceil(E / (_NS * _L * _G)) * (_NS * _L * _G)
    n_groups = epad // (_NS * _L * _G)
    ones = jnp.ones((n, 1), jnp.float32)
    z15 = jnp.zeros((n, 15), jnp.float32)
    x_aug = jnp.stack(
        [jnp.concatenate([x[:, :64], ones, z15], axis=1),
         jnp.concatenate([x[:, 64:], ones, z15], axis=1)])
    x_aug = jnp.pad(x_aug, ((0, 0), (0, n_tab - n), (0, 0)))
    fill = jnp.full((epad - E,), n, jnp.int32)
    srcp = jnp.concatenate([src.astype(jnp.int32), fill]).reshape(
        _NS, n_groups, _G, _L)
    dstp = jnp.concatenate([dst.astype(jnp.int32), fill]).reshape(
        _NS, n_groups, _G, _L)
    zeros = jnp.zeros((_L, 80), jnp.float32)
    fn = pl.kernel(
        functools.partial(_sc_segsum_body, n_tab=n_tab, n_groups=n_groups),
        out_type=jax.ShapeDtypeStruct((_NC, n_tab, 80), jnp.float32),
        mesh=plsc.VectorSubcoreMesh(core_axis_name="c", subcore_axis_name="s"),
        compiler_params=pltpu.CompilerParams(use_tc_tiling_on_sc=False),
        scratch_types=[
            pltpu.VMEM_SHARED((n_tab, 80), jnp.float32),
            pltpu.VMEM_SHARED((n_tab, 80), jnp.float32),
            pltpu.VMEM((_L, 80), jnp.float32),
            pltpu.VMEM((_L, 80), jnp.float32),
            pltpu.VMEM((_G, _L), jnp.int32),
            pltpu.VMEM((_G, _L), jnp.int32),
            pltpu.SemaphoreType.DMA,
        ],
    )
    out = fn(x_aug, srcp, dstp, zeros)
    sums = jnp.concatenate([out[0, :n, :64], out[1, :n, :64]], axis=1)
    return sums, out[0, :n, 64:65]


def _mm_body(x_ref, w_ref, o_ref, *, relu_in):
    x = x_ref[...]
    if relu_in:
        x = jnp.maximum(x, 0.0)
    o_ref[...] = jnp.dot(x, w_ref[...], preferred_element_type=jnp.float32, precision=lax.Precision.HIGHEST)


def _mm_scale_body(x_ref, s_ref, w_ref, o_ref, *, relu_in):
    x = x_ref[...]
    if relu_in:
        x = jnp.maximum(x, 0.0)
    x = x * s_ref[...]
    o_ref[...] = jnp.dot(x, w_ref[...], preferred_element_type=jnp.float32, precision=lax.Precision.HIGHEST)


def _mm(x, w, scale=None, relu_in=False, bm=512):
    n, d = x.shape
    npad = math.ceil(n / bm) * bm
    if npad != n:
        x = jnp.pad(x, ((0, npad - n), (0, 0)))
        if scale is not None:
            scale = jnp.pad(scale, ((0, npad - n), (0, 0)))
    grid = (npad // bm,)
    out_shape = jax.ShapeDtypeStruct((npad, w.shape[1]), jnp.float32)
    if scale is None:
        fn = pl.pallas_call(
            functools.partial(_mm_body, relu_in=relu_in),
            grid=grid,
            in_specs=[
                pl.BlockSpec((bm, d), lambda i: (i, 0)),
                pl.BlockSpec((d, w.shape[1]), lambda i: (0, 0)),
            ],
            out_specs=pl.BlockSpec((bm, w.shape[1]), lambda i: (i, 0)),
            out_shape=out_shape,
        )
        y = fn(x, w)
    else:
        fn = pl.pallas_call(
            functools.partial(_mm_scale_body, relu_in=relu_in),
            grid=grid,
            in_specs=[
                pl.BlockSpec((bm, d), lambda i: (i, 0)),
                pl.BlockSpec((bm, 1), lambda i: (i, 0)),
                pl.BlockSpec((d, w.shape[1]), lambda i: (0, 0)),
            ],
            out_specs=pl.BlockSpec((bm, w.shape[1]), lambda i: (i, 0)),
            out_shape=out_shape,
        )
        y = fn(x, scale, w)
    return y[:n]


def _mmoe_body(x_ref, ew_ref, eb_ref, gw_ref, tw1_ref, tb1_ref, tw2_ref,
               tb2_ref, o_ref):
    x = x_ref[...]
    eo = []
    for k in range(6):
        pre = jnp.dot(x, ew_ref[k], preferred_element_type=jnp.float32, precision=lax.Precision.HIGHEST)
        eo.append(jnp.maximum(pre + eb_ref[k][None, :], 0.0))
    outs = []
    for t in range(2):
        logits = jnp.dot(x, gw_ref[t], preferred_element_type=jnp.float32, precision=lax.Precision.HIGHEST)
        logits = logits - jnp.max(logits, axis=-1, keepdims=True)
        eg = jnp.exp(logits)
        g = eg / jnp.sum(eg, axis=-1, keepdims=True)
        mixed = g[:, 0:1] * eo[0]
        for k in range(1, 6):
            mixed = mixed + g[:, k:k + 1] * eo[k]
        t1 = jnp.dot(mixed, tw1_ref[t], preferred_element_type=jnp.float32, precision=lax.Precision.HIGHEST)
        t1 = jnp.maximum(t1 + tb1_ref[t][None, :], 0.0)
        t2 = jnp.dot(t1, tw2_ref[t], preferred_element_type=jnp.float32, precision=lax.Precision.HIGHEST)
        outs.append(t2[:, 0] + tb2_ref[t][0])
    o_ref[...] = jnp.stack(outs, axis=-1)


def _mmoe(x, expert_W, expert_b, gate_W, tower_W1, tower_b1, tower_W2,
          tower_b2, bm=512):
    m, es = x.shape
    assert m % bm == 0
    grid = (m // bm,)
    full = lambda shp: pl.BlockSpec(shp, lambda i: tuple(0 for _ in shp))
    fn = pl.pallas_call(
        _mmoe_body,
        grid=grid,
        in_specs=[
            pl.BlockSpec((bm, es), lambda i: (i, 0)),
            full(expert_W.shape), full(expert_b.shape), full(gate_W.shape),
            full(tower_W1.shape), full(tower_b1.shape), full(tower_W2.shape),
            full(tower_b2.shape),
        ],
        out_specs=pl.BlockSpec((bm, 2), lambda i: (i, 0)),
        out_shape=jax.ShapeDtypeStruct((m, 2), jnp.float32),
    )
    return fn(x, expert_W, expert_b, gate_W, tower_W1, tower_b1, tower_W2,
              tower_b2)


def kernel(target_user, item_sample, user_sample, embed, embed_ui, embed_pi,
           embed_u, W_hg, W_g1, W_g2, hyper_src, hyper_dst, ii_src, ii_dst,
           pi_src, pi_dst, ip_src, ip_dst, expert_W, expert_b, gate_W,
           tower_W1, tower_b1, tower_W2, tower_b2):
    N, U, I = N_NODES, U_NUM, I_NUM

    t0 = _mm(embed, W_hg)
    s_h, deg_h = _sc_segsum(t0, hyper_src, hyper_dst, N)
    dinv_hyper = 1.0 / jnp.clip(deg_h, 1.0)

    def gcn(x_pre_sum, x_dinv, src, dst, n, relu_in):
        t1 = _mm(x_pre_sum, W_g1, scale=x_dinv, relu_in=relu_in)
        s1, deg = _sc_segsum(t1, src, dst, n)
        dinv = 1.0 / jnp.clip(deg, 1.0)
        t2 = _mm(s1, W_g2, scale=dinv, relu_in=True)
        s2, _ = _sc_segsum(t2, src, dst, n)
        return s2 * dinv

    ones_n = jnp.ones((N, 1), jnp.float32)
    ones_u = jnp.ones((U, 1), jnp.float32)
    init_item_h = gcn(s_h, dinv_hyper, ii_src, ii_dst, N, True)
    part_item_h = gcn(s_h, dinv_hyper, pi_src, pi_dst, N, True)
    init_part_h = gcn(s_h[:U], dinv_hyper[:U], ip_src, ip_dst, U, True)
    init_item_g = gcn(embed_ui, ones_n, ii_src, ii_dst, N, False)
    part_item_g = gcn(embed_pi, ones_n, pi_src, pi_dst, N, False)
    init_part_g = gcn(embed_u, ones_u, ip_src, ip_dst, U, False)

    init_item_embed = jnp.concatenate((init_item_h, init_item_g), axis=1)
    part_item_embed = jnp.concatenate((part_item_h, part_item_g), axis=1)
    init_part_embed = jnp.concatenate((init_part_h, init_part_g), axis=1)

    init_item_type = init_item_embed[:U]
    init_part_type = init_part_embed[:U]
    part_item_type = part_item_embed[:U]
    part_init_type = init_part_embed[:U]
    item_init_type = init_item_embed[U:U + I]
    item_part_type = part_item_embed[U:U + I]

    allp = jnp.mean(jnp.concatenate((part_item_type, part_init_type), axis=1),
                    axis=0, keepdims=True)
    tu = jnp.concatenate((init_item_type[target_user][:, None, :],
                          init_part_type[target_user][:, None, :]), axis=2)
    B, Si = item_sample.shape
    isf = item_sample.reshape(-1)
    item_sample_embed = jnp.concatenate(
        (item_init_type[isf].reshape(B, Si, -1),
         item_part_type[isf].reshape(B, Si, -1)), axis=2)
    Sp = user_sample.shape[1]
    usf = user_sample.reshape(-1)
    user_sample_embed = jnp.concatenate(
        (part_item_type[usf].reshape(B, Sp, -1),
         part_init_type[usf].reshape(B, Sp, -1)), axis=2)
    true_item = item_sample_embed[:, 0:1]
    users1 = jnp.tile(tu, (1, Si, 1))
    users2 = jnp.tile(tu, (1, Sp, 1))
    true_is = jnp.tile(true_item, (1, Sp, 1))
    allp_b = jnp.tile(allp[None], (B, Si, 1))
    u_isample_p = jnp.concatenate((users1, item_sample_embed, allp_b), axis=2)
    u_i_psample = jnp.concatenate((users2, true_is, user_sample_embed),
                                  axis=2)
    u_i_p = jnp.concatenate((u_isample_p, u_i_psample), axis=1)
    bs, ss, es = u_i_p.shape
    x = u_i_p.reshape(bs * ss, es)

    o = _mmoe(x, expert_W, expert_b, gate_W, tower_W1, tower_b1, tower_W2,
              tower_b2)
    output1 = o[:, 0].reshape(bs, ss)
    output2 = o[:, 1].reshape(bs, ss)
    loc = ss // 2
    task1_score = output1[:, :loc]
    task2_score = output2[:, loc:]

    def bpr(inp):
        return jnp.mean(-jax.nn.log_sigmoid(inp[:, 0:1] - inp[:, 1:]),
                        axis=-1)

    bprloss = 0.2 * bpr(task1_score[:, 0:5]) + bpr(task2_score[:, 0:5])
    truelabels = jnp.ones((bs, ss), jnp.float32).at[:, 1:loc].set(0.0)
    listloss = -jnp.sum(jax.nn.softmax(truelabels, axis=1)
                        * jnp.log(jax.nn.softmax(output1, axis=1)), axis=1)
    loss = bprloss + 0.3 * listloss + bpr(output2[:, :loc])
    return loss, task1_score, task2_score


# speedup vs baseline: 2.4145x; 1.0667x over previous
import functools
import math

import jax
import jax.numpy as jnp
from jax import lax
from jax.experimental import pallas as pl
from jax.experimental.pallas import tpu as pltpu
from jax.experimental.pallas import tpu_sc as plsc

N_NODES, U_NUM, I_NUM = 3000 + 7000, 3000, 7000
_NS, _NC, _L = 16, 2, 128


_G = 8


def _sc_pass_body(x_ref, src_ref, dst_ref, z_ref, out_ref,
                  outs, rows, src_v, dst_v, gsem, ssem,
                  *, w, n_tab, n_groups):
    c = lax.axis_index("c")
    s = lax.axis_index("s")
    r_t = n_tab // _NS
    row0 = s * r_t
    n_slabs = r_t // _L
    pltpu.sync_copy(z_ref, rows.at[0])
    for v in range(n_slabs):
        pltpu.sync_copy(rows.at[0], outs.at[pl.ds(row0 + v * _L, _L)])
    plsc.subcore_barrier()

    @pl.loop(0, n_groups)
    def _grp(g):
        pltpu.sync_copy(src_ref.at[s, g], src_v)
        pltpu.sync_copy(dst_ref.at[s, g], dst_v)
        gh = [None] * _G
        sh = [None] * _G
        gh[0] = pltpu.async_copy(x_ref.at[c].at[src_v.at[0]], rows.at[0], gsem)
        for b in range(_G):
            gh[b].wait()
            if b >= 1:
                sh[b - 1].wait()
            if b + 1 < _G:
                gh[b + 1] = pltpu.async_copy(
                    x_ref.at[c].at[src_v.at[b + 1]], rows.at[(b + 1) % 2], gsem)
            sh[b] = pltpu.async_copy(rows.at[b % 2], outs.at[dst_v.at[b]],
                                     ssem, add=True)
        sh[_G - 1].wait()

    plsc.subcore_barrier()
    for v in range(n_slabs):
        pltpu.sync_copy(outs.at[pl.ds(row0 + v * _L, _L)], rows.at[0])
        pltpu.sync_copy(rows.at[0], out_ref.at[c, pl.ds(row0 + v * _L, _L)])


def _sc_pass(xa, xb, src, dst, n):
    w = xa.shape[1]
    n_tab = math.ceil((n + 8) / (_NS * _L)) * (_NS * _L)
    E = src.shape[0]
    epad = math.ceil(E / (_NS * _L * _G)) * (_NS * _L * _G)
    n_groups = epad // (_NS * _L * _G)
    x2 = jnp.pad(jnp.stack([xa, xb]), ((0, 0), (0, n_tab - n), (0, 0)))
    fill = jnp.full((epad - E,), n, jnp.int32)
    srcp = jnp.concatenate([src.astype(jnp.int32), fill]).reshape(
        _NS, n_groups, _G, _L)
    dstp = jnp.concatenate([dst.astype(jnp.int32), fill]).reshape(
        _NS, n_groups, _G, _L)
    zeros = jnp.zeros((_L, w), jnp.float32)
    fn = pl.kernel(
        functools.partial(_sc_pass_body, w=w, n_tab=n_tab, n_groups=n_groups),
        out_type=jax.ShapeDtypeStruct((_NC, n_tab, w), jnp.float32),
        mesh=plsc.VectorSubcoreMesh(core_axis_name="c", subcore_axis_name="s"),
        compiler_params=pltpu.CompilerParams(use_tc_tiling_on_sc=False),
        scratch_types=[
            pltpu.VMEM_SHARED((n_tab, w), jnp.float32),
            pltpu.VMEM((2, _L, w), jnp.float32),
            pltpu.VMEM((_G, _L), jnp.int32),
            pltpu.VMEM((_G, _L), jnp.int32),
            pltpu.SemaphoreType.DMA,
            pltpu.SemaphoreType.DMA,
        ],
    )
    out = fn(x2, srcp, dstp, zeros)
    return out[0, :n], out[1, :n]


def _sc_deg_body(dst_ref, one_ref, z_ref, out_ref,
                 degs, onev, stage, dst_v, ssem, *, n_tab, n_groups):
    c = lax.axis_index("c")
    s = lax.axis_index("s")
    r_t = n_tab // _NS
    row0 = s * r_t
    n_slabs = r_t // _L
    pltpu.sync_copy(one_ref, onev)
    pltpu.sync_copy(z_ref, stage)
    for v in range(n_slabs):
        pltpu.sync_copy(stage, degs.at[pl.ds(row0 + v * _L, _L)])
    plsc.subcore_barrier()

    @pl.loop(0, n_groups)
    def _grp(g):
        pltpu.sync_copy(dst_ref.at[c, s, g], dst_v)
        sh = []
        for b in range(_G):
            sh.append(pltpu.async_copy(onev, degs.at[dst_v.at[b]], ssem,
                                       add=True))
        for h in sh:
            h.wait()

    plsc.subcore_barrier()
    for v in range(n_slabs):
        pltpu.sync_copy(degs.at[pl.ds(row0 + v * _L, _L)], stage)
        pltpu.sync_copy(stage, out_ref.at[c, pl.ds(row0 + v * _L, _L)])


def _sc_deg(dsts_ns):
    n_big = max(n for _, n in dsts_ns)
    base = math.ceil((n_big + 8) / (_NS * _L)) * (_NS * _L)
    offs = []
    o = 0
    for _, n in dsts_ns:
        offs.append(o)
        o += base
    cap = _NS * _L * _G
    etot = sum(d.shape[0] for d, _ in dsts_ns)
    n_groups = math.ceil(etot / (_NC * cap))
    epad = n_groups * _NC * cap
    n_tab = o
    junk = n_tab - 1
    parts = [d.astype(jnp.int32) + off for (d, _), off in zip(dsts_ns, offs)]
    parts.append(jnp.full((epad - etot,), junk, jnp.int32))
    dstp = jnp.concatenate(parts).reshape(_NC, _NS, n_groups, _G, _L)
    ones = jnp.ones((_L, 16), jnp.float32)
    zeros = jnp.zeros((_L, 16), jnp.float32)
    fn = pl.kernel(
        functools.partial(_sc_deg_body, n_tab=n_tab, n_groups=n_groups),
        out_type=jax.ShapeDtypeStruct((_NC, n_tab, 16), jnp.float32),
        mesh=plsc.VectorSubcoreMesh(core_axis_name="c", subcore_axis_name="s"),
        compiler_params=pltpu.CompilerParams(use_tc_tiling_on_sc=False),
        scratch_types=[
            pltpu.VMEM_SHARED((n_tab, 16), jnp.float32),
            pltpu.VMEM((_L, 16), jnp.float32),
            pltpu.VMEM((_L, 16), jnp.float32),
            pltpu.VMEM((_G, _L), jnp.int32),
            pltpu.SemaphoreType.DMA,
        ],
    )
    out = fn(dstp, ones, zeros)
    deg = out[0, :, 0:1] + out[1, :, 0:1]
    return [deg[off:off + n] for (_, n), off in zip(dsts_ns, offs)]


def _mm_body(x_ref, w_ref, o_ref, *, relu_in):
    x = x_ref[...]
    if relu_in:
        x = jnp.maximum(x, 0.0)
    o_ref[...] = jnp.dot(x, w_ref[...], preferred_element_type=jnp.float32)


def _mm_scale_body(x_ref, s_ref, w_ref, o_ref, *, relu_in):
    x = x_ref[...]
    if relu_in:
        x = jnp.maximum(x, 0.0)
    x = x * s_ref[...]
    o_ref[...] = jnp.dot(x, w_ref[...], preferred_element_type=jnp.float32)


def _mm(x, w, scale=None, relu_in=False, bm=512):
    n, d = x.shape
    npad = math.ceil(n / bm) * bm
    if npad != n:
        x = jnp.pad(x, ((0, npad - n), (0, 0)))
        if scale is not None:
            scale = jnp.pad(scale, ((0, npad - n), (0, 0)))
    grid = (npad // bm,)
    out_shape = jax.ShapeDtypeStruct((npad, w.shape[1]), jnp.float32)
    if scale is None:
        fn = pl.pallas_call(
            functools.partial(_mm_body, relu_in=relu_in),
            grid=grid,
            in_specs=[
                pl.BlockSpec((bm, d), lambda i: (i, 0)),
                pl.BlockSpec((d, w.shape[1]), lambda i: (0, 0)),
            ],
            out_specs=pl.BlockSpec((bm, w.shape[1]), lambda i: (i, 0)),
            out_shape=out_shape,
        )
        y = fn(x, w)
    else:
        fn = pl.pallas_call(
            functools.partial(_mm_scale_body, relu_in=relu_in),
            grid=grid,
            in_specs=[
                pl.BlockSpec((bm, d), lambda i: (i, 0)),
                pl.BlockSpec((bm, 1), lambda i: (i, 0)),
                pl.BlockSpec((d, w.shape[1]), lambda i: (0, 0)),
            ],
            out_specs=pl.BlockSpec((bm, w.shape[1]), lambda i: (i, 0)),
            out_shape=out_shape,
        )
        y = fn(x, scale, w)
    return y[:n]


def _mmoe_body(x_ref, ew_ref, eb_ref, gw_ref, tw1_ref, tb1_ref, tw2_ref,
               tb2_ref, o_ref):
    x = x_ref[...]
    eo = []
    for k in range(6):
        pre = jnp.dot(x, ew_ref[k], preferred_element_type=jnp.float32)
        eo.append(jnp.maximum(pre + eb_ref[k][None, :], 0.0))
    outs = []
    for t in range(2):
        logits = jnp.dot(x, gw_ref[t], preferred_element_type=jnp.float32)
        logits = logits - jnp.max(logits, axis=-1, keepdims=True)
        eg = jnp.exp(logits)
        g = eg / jnp.sum(eg, axis=-1, keepdims=True)
        mixed = g[:, 0:1] * eo[0]
        for k in range(1, 6):
            mixed = mixed + g[:, k:k + 1] * eo[k]
        t1 = jnp.dot(mixed, tw1_ref[t], preferred_element_type=jnp.float32)
        t1 = jnp.maximum(t1 + tb1_ref[t][None, :], 0.0)
        t2 = jnp.dot(t1, tw2_ref[t], preferred_element_type=jnp.float32)
        outs.append(t2[:, 0] + tb2_ref[t][0])
    o_ref[...] = jnp.stack(outs, axis=-1)


def _mmoe(x, expert_W, expert_b, gate_W, tower_W1, tower_b1, tower_W2,
          tower_b2, bm=512):
    m, es = x.shape
    assert m % bm == 0
    grid = (m // bm,)
    full = lambda shp: pl.BlockSpec(shp, lambda i: tuple(0 for _ in shp))
    fn = pl.pallas_call(
        _mmoe_body,
        grid=grid,
        in_specs=[
            pl.BlockSpec((bm, es), lambda i: (i, 0)),
            full(expert_W.shape), full(expert_b.shape), full(gate_W.shape),
            full(tower_W1.shape), full(tower_b1.shape), full(tower_W2.shape),
            full(tower_b2.shape),
        ],
        out_specs=pl.BlockSpec((bm, 2), lambda i: (i, 0)),
        out_shape=jax.ShapeDtypeStruct((m, 2), jnp.float32),
    )
    return fn(x, expert_W, expert_b, gate_W, tower_W1, tower_b1, tower_W2,
              tower_b2)


def kernel(target_user, item_sample, user_sample, embed, embed_ui, embed_pi,
           embed_u, W_hg, W_g1, W_g2, hyper_src, hyper_dst, ii_src, ii_dst,
           pi_src, pi_dst, ip_src, ip_dst, expert_W, expert_b, gate_W,
           tower_W1, tower_b1, tower_W2, tower_b2):
    N, U, I = N_NODES, U_NUM, I_NUM

    deg_ii, deg_pi, deg_ip, deg_hy = _sc_deg(
        [(ii_dst, N), (pi_dst, N), (ip_dst, U), (hyper_dst, N)])
    dinv_ii = 1.0 / jnp.clip(deg_ii, 1.0)
    dinv_pi = 1.0 / jnp.clip(deg_pi, 1.0)
    dinv_ip = 1.0 / jnp.clip(deg_ip, 1.0)
    dinv_hyper = 1.0 / jnp.clip(deg_hy, 1.0)

    t0 = _mm(embed, W_hg)
    sh_a, sh_b = _sc_pass(t0[:, :64], t0[:, 64:], hyper_src, hyper_dst, N)
    s_h = jnp.concatenate([sh_a, sh_b], axis=1)

    def gcn2(xh_pre, xh_dinv, relu_h, xg, src, dst, dinv, n):
        t1h = _mm(xh_pre, W_g1, scale=xh_dinv, relu_in=relu_h)
        t1g = _mm(xg, W_g1)
        s1h, s1g = _sc_pass(t1h, t1g, src, dst, n)
        t2h = _mm(s1h, W_g2, scale=dinv, relu_in=True)
        t2g = _mm(s1g, W_g2, scale=dinv, relu_in=True)
        s2h, s2g = _sc_pass(t2h, t2g, src, dst, n)
        return s2h * dinv, s2g * dinv

    init_item_h, init_item_g = gcn2(s_h, dinv_hyper, True, embed_ui,
                                    ii_src, ii_dst, dinv_ii, N)
    part_item_h, part_item_g = gcn2(s_h, dinv_hyper, True, embed_pi,
                                    pi_src, pi_dst, dinv_pi, N)
    init_part_h, init_part_g = gcn2(s_h[:U], dinv_hyper[:U], True, embed_u,
                                    ip_src, ip_dst, dinv_ip, U)

    init_item_embed = jnp.concatenate((init_item_h, init_item_g), axis=1)
    part_item_embed = jnp.concatenate((part_item_h, part_item_g), axis=1)
    init_part_embed = jnp.concatenate((init_part_h, init_part_g), axis=1)

    init_item_type = init_item_embed[:U]
    init_part_type = init_part_embed[:U]
    part_item_type = part_item_embed[:U]
    part_init_type = init_part_embed[:U]
    item_init_type = init_item_embed[U:U + I]
    item_part_type = part_item_embed[U:U + I]

    allp = jnp.mean(jnp.concatenate((part_item_type, part_init_type), axis=1),
                    axis=0, keepdims=True)
    tu = jnp.concatenate((init_item_type[target_user][:, None, :],
                          init_part_type[target_user][:, None, :]), axis=2)
    B, Si = item_sample.shape
    isf = item_sample.reshape(-1)
    item_sample_embed = jnp.concatenate(
        (item_init_type[isf].reshape(B, Si, -1),
         item_part_type[isf].reshape(B, Si, -1)), axis=2)
    Sp = user_sample.shape[1]
    usf = user_sample.reshape(-1)
    user_sample_embed = jnp.concatenate(
        (part_item_type[usf].reshape(B, Sp, -1),
         part_init_type[usf].reshape(B, Sp, -1)), axis=2)
    true_item = item_sample_embed[:, 0:1]
    users1 = jnp.tile(tu, (1, Si, 1))
    users2 = jnp.tile(tu, (1, Sp, 1))
    true_is = jnp.tile(true_item, (1, Sp, 1))
    allp_b = jnp.tile(allp[None], (B, Si, 1))
    u_isample_p = jnp.concatenate((users1, item_sample_embed, allp_b), axis=2)
    u_i_psample = jnp.concatenate((users2, true_is, user_sample_embed),
                                  axis=2)
    u_i_p = jnp.concatenate((u_isample_p, u_i_psample), axis=1)
    bs, ss, es = u_i_p.shape
    x = u_i_p.reshape(bs * ss, es)

    o = _mmoe(x, expert_W, expert_b, gate_W, tower_W1, tower_b1, tower_W2,
              tower_b2)
    output1 = o[:, 0].reshape(bs, ss)
    output2 = o[:, 1].reshape(bs, ss)
    loc = ss // 2
    task1_score = output1[:, :loc]
    task2_score = output2[:, loc:]

    def bpr(inp):
        return jnp.mean(-jax.nn.log_sigmoid(inp[:, 0:1] - inp[:, 1:]),
                        axis=-1)

    bprloss = 0.2 * bpr(task1_score[:, 0:5]) + bpr(task2_score[:, 0:5])
    truelabels = jnp.ones((bs, ss), jnp.float32).at[:, 1:loc].set(0.0)
    listloss = -jnp.sum(jax.nn.softmax(truelabels, axis=1)
                        * jnp.log(jax.nn.softmax(output1, axis=1)), axis=1)
    loss = bprloss + 0.3 * listloss + bpr(output2[:, :loc])
    return loss, task1_score, task2_score


# speedup vs baseline: 2.5031x; 1.0367x over previous
import functools
import math

import jax
import jax.numpy as jnp
from jax import lax
from jax.experimental import pallas as pl
from jax.experimental.pallas import tpu as pltpu
from jax.experimental.pallas import tpu_sc as plsc

N_NODES, U_NUM, I_NUM = 3000 + 7000, 3000, 7000
_NS, _NC, _L = 16, 2, 128


_G = 8


def _sc_pass_body(x_ref, src_ref, dst_ref, z_ref, out_ref,
                  outs, rows, src_v, dst_v, gsem, ssem, isem,
                  *, w, n_tab, n_groups):
    c = lax.axis_index("c")
    s = lax.axis_index("s")
    r_t = n_tab // _NS
    row0 = s * r_t
    n_slabs = r_t // _L
    pltpu.sync_copy(z_ref, rows.at[0])
    for v in range(n_slabs):
        pltpu.sync_copy(rows.at[0], outs.at[pl.ds(row0 + v * _L, _L)])
    plsc.subcore_barrier()

    pltpu.async_copy(src_ref.at[s, 0], src_v.at[0], isem)
    pltpu.async_copy(dst_ref.at[s, 0], dst_v.at[0], isem)

    @pl.loop(0, n_groups)
    def _grp(g):
        p = g % 2
        pltpu.make_async_copy(src_ref.at[s, g], src_v.at[p], isem).wait()
        pltpu.make_async_copy(dst_ref.at[s, g], dst_v.at[p], isem).wait()

        @pl.when(g + 1 < n_groups)
        def _prefetch():
            pltpu.async_copy(src_ref.at[s, g + 1], src_v.at[1 - p], isem)
            pltpu.async_copy(dst_ref.at[s, g + 1], dst_v.at[1 - p], isem)

        gh = [None] * _G
        sh = [None] * _G
        gh[0] = pltpu.async_copy(x_ref.at[c].at[src_v.at[p, 0]], rows.at[0],
                                 gsem)
        for b in range(_G):
            gh[b].wait()
            if b >= 1:
                sh[b - 1].wait()
            if b + 1 < _G:
                gh[b + 1] = pltpu.async_copy(
                    x_ref.at[c].at[src_v.at[p, b + 1]], rows.at[(b + 1) % 2],
                    gsem)
            sh[b] = pltpu.async_copy(rows.at[b % 2], outs.at[dst_v.at[p, b]],
                                     ssem, add=True)
        sh[_G - 1].wait()

    plsc.subcore_barrier()
    for v in range(n_slabs):
        pltpu.sync_copy(outs.at[pl.ds(row0 + v * _L, _L)], rows.at[0])
        pltpu.sync_copy(rows.at[0], out_ref.at[c, pl.ds(row0 + v * _L, _L)])


def _sc_pass(xa, xb, src, dst, n):
    w = xa.shape[1]
    n_tab = math.ceil((n + 8) / (_NS * _L)) * (_NS * _L)
    E = src.shape[0]
    epad = math.ceil(E / (_NS * _L * _G)) * (_NS * _L * _G)
    n_groups = epad // (_NS * _L * _G)
    x2 = jnp.pad(jnp.stack([xa, xb]), ((0, 0), (0, n_tab - n), (0, 0)))
    fill = jnp.full((epad - E,), n, jnp.int32)
    srcp = jnp.concatenate([src.astype(jnp.int32), fill]).reshape(
        _NS, n_groups, _G, _L)
    dstp = jnp.concatenate([dst.astype(jnp.int32), fill]).reshape(
        _NS, n_groups, _G, _L)
    zeros = jnp.zeros((_L, w), jnp.float32)
    fn = pl.kernel(
        functools.partial(_sc_pass_body, w=w, n_tab=n_tab, n_groups=n_groups),
        out_type=jax.ShapeDtypeStruct((_NC, n_tab, w), jnp.float32),
        mesh=plsc.VectorSubcoreMesh(core_axis_name="c", subcore_axis_name="s"),
        compiler_params=pltpu.CompilerParams(use_tc_tiling_on_sc=False),
        scratch_types=[
            pltpu.VMEM_SHARED((n_tab, w), jnp.float32),
            pltpu.VMEM((2, _L, w), jnp.float32),
            pltpu.VMEM((2, _G, _L), jnp.int32),
            pltpu.VMEM((2, _G, _L), jnp.int32),
            pltpu.SemaphoreType.DMA,
            pltpu.SemaphoreType.DMA,
            pltpu.SemaphoreType.DMA,
        ],
    )
    out = fn(x2, srcp, dstp, zeros)
    return out[0, :n], out[1, :n]


_GD = 32


def _sc_deg_body(dst_ref, one_ref, z_ref, out_ref,
                 degs, onev, stage, dst_v, ssem, isem, *, n_tab, n_groups):
    c = lax.axis_index("c")
    s = lax.axis_index("s")
    r_t = n_tab // _NS
    row0 = s * r_t
    n_slabs = r_t // _L
    pltpu.sync_copy(one_ref, onev)
    pltpu.sync_copy(z_ref, stage)
    for v in range(n_slabs):
        pltpu.sync_copy(stage, degs.at[pl.ds(row0 + v * _L, _L)])
    plsc.subcore_barrier()
    pltpu.async_copy(dst_ref.at[c, s, 0], dst_v.at[0], isem)

    @pl.loop(0, n_groups)
    def _grp(g):
        p = g % 2
        pltpu.make_async_copy(dst_ref.at[c, s, g], dst_v.at[p], isem).wait()

        @pl.when(g + 1 < n_groups)
        def _prefetch():
            pltpu.async_copy(dst_ref.at[c, s, g + 1], dst_v.at[1 - p], isem)

        sh = []
        for b in range(_GD):
            sh.append(pltpu.async_copy(onev, degs.at[dst_v.at[p, b]], ssem,
                                       add=True))
        for h in sh:
            h.wait()

    plsc.subcore_barrier()
    for v in range(n_slabs):
        pltpu.sync_copy(degs.at[pl.ds(row0 + v * _L, _L)], stage)
        pltpu.sync_copy(stage, out_ref.at[c, pl.ds(row0 + v * _L, _L)])


def _sc_deg(dsts_ns):
    n_big = max(n for _, n in dsts_ns)
    base = math.ceil((n_big + 8) / (_NS * _L)) * (_NS * _L)
    offs = []
    o = 0
    for _, n in dsts_ns:
        offs.append(o)
        o += base
    cap = _NS * _L * _GD
    etot = sum(d.shape[0] for d, _ in dsts_ns)
    n_groups = math.ceil(etot / (_NC * cap))
    epad = n_groups * _NC * cap
    n_tab = o
    junk = n_tab - 1
    parts = [d.astype(jnp.int32) + off for (d, _), off in zip(dsts_ns, offs)]
    parts.append(jnp.full((epad - etot,), junk, jnp.int32))
    dstp = jnp.concatenate(parts).reshape(_NC, _NS, n_groups, _GD, _L)
    ones = jnp.ones((_L, 16), jnp.float32)
    zeros = jnp.zeros((_L, 16), jnp.float32)
    fn = pl.kernel(
        functools.partial(_sc_deg_body, n_tab=n_tab, n_groups=n_groups),
        out_type=jax.ShapeDtypeStruct((_NC, n_tab, 16), jnp.float32),
        mesh=plsc.VectorSubcoreMesh(core_axis_name="c", subcore_axis_name="s"),
        compiler_params=pltpu.CompilerParams(use_tc_tiling_on_sc=False),
        scratch_types=[
            pltpu.VMEM_SHARED((n_tab, 16), jnp.float32),
            pltpu.VMEM((_L, 16), jnp.float32),
            pltpu.VMEM((_L, 16), jnp.float32),
            pltpu.VMEM((2, _GD, _L), jnp.int32),
            pltpu.SemaphoreType.DMA,
            pltpu.SemaphoreType.DMA,
        ],
    )
    out = fn(dstp, ones, zeros)
    deg = out[0, :, 0:1] + out[1, :, 0:1]
    return [deg[off:off + n] for (_, n), off in zip(dsts_ns, offs)]


def _mm_body(x_ref, w_ref, o_ref, *, relu_in):
    x = x_ref[...]
    if relu_in:
        x = jnp.maximum(x, 0.0)
    o_ref[...] = jnp.dot(x, w_ref[...], preferred_element_type=jnp.float32)


def _mm_scale_body(x_ref, s_ref, w_ref, o_ref, *, relu_in):
    x = x_ref[...]
    if relu_in:
        x = jnp.maximum(x, 0.0)
    x = x * s_ref[...]
    o_ref[...] = jnp.dot(x, w_ref[...], preferred_element_type=jnp.float32)


def _mm(x, w, scale=None, relu_in=False, bm=512):
    n, d = x.shape
    npad = math.ceil(n / bm) * bm
    if npad != n:
        x = jnp.pad(x, ((0, npad - n), (0, 0)))
        if scale is not None:
            scale = jnp.pad(scale, ((0, npad - n), (0, 0)))
    grid = (npad // bm,)
    out_shape = jax.ShapeDtypeStruct((npad, w.shape[1]), jnp.float32)
    if scale is None:
        fn = pl.pallas_call(
            functools.partial(_mm_body, relu_in=relu_in),
            grid=grid,
            in_specs=[
                pl.BlockSpec((bm, d), lambda i: (i, 0)),
                pl.BlockSpec((d, w.shape[1]), lambda i: (0, 0)),
            ],
            out_specs=pl.BlockSpec((bm, w.shape[1]), lambda i: (i, 0)),
            out_shape=out_shape,
        )
        y = fn(x, w)
    else:
        fn = pl.pallas_call(
            functools.partial(_mm_scale_body, relu_in=relu_in),
            grid=grid,
            in_specs=[
                pl.BlockSpec((bm, d), lambda i: (i, 0)),
                pl.BlockSpec((bm, 1), lambda i: (i, 0)),
                pl.BlockSpec((d, w.shape[1]), lambda i: (0, 0)),
            ],
            out_specs=pl.BlockSpec((bm, w.shape[1]), lambda i: (i, 0)),
            out_shape=out_shape,
        )
        y = fn(x, scale, w)
    return y[:n]


def _mmoe_body(x_ref, ew_ref, eb_ref, gw_ref, tw1_ref, tb1_ref, tw2_ref,
               tb2_ref, o_ref):
    x = x_ref[...]
    eo = []
    for k in range(6):
        pre = jnp.dot(x, ew_ref[k], preferred_element_type=jnp.float32)
        eo.append(jnp.maximum(pre + eb_ref[k][None, :], 0.0))
    outs = []
    for t in range(2):
        logits = jnp.dot(x, gw_ref[t], preferred_element_type=jnp.float32)
        logits = logits - jnp.max(logits, axis=-1, keepdims=True)
        eg = jnp.exp(logits)
        g = eg / jnp.sum(eg, axis=-1, keepdims=True)
        mixed = g[:, 0:1] * eo[0]
        for k in range(1, 6):
            mixed = mixed + g[:, k:k + 1] * eo[k]
        t1 = jnp.dot(mixed, tw1_ref[t], preferred_element_type=jnp.float32)
        t1 = jnp.maximum(t1 + tb1_ref[t][None, :], 0.0)
        t2 = jnp.dot(t1, tw2_ref[t], preferred_element_type=jnp.float32)
        outs.append(t2[:, 0] + tb2_ref[t][0])
    o_ref[...] = jnp.stack(outs, axis=-1)


def _mmoe(x, expert_W, expert_b, gate_W, tower_W1, tower_b1, tower_W2,
          tower_b2, bm=512):
    m, es = x.shape
    assert m % bm == 0
    grid = (m // bm,)
    full = lambda shp: pl.BlockSpec(shp, lambda i: tuple(0 for _ in shp))
    fn = pl.pallas_call(
        _mmoe_body,
        grid=grid,
        in_specs=[
            pl.BlockSpec((bm, es), lambda i: (i, 0)),
            full(expert_W.shape), full(expert_b.shape), full(gate_W.shape),
            full(tower_W1.shape), full(tower_b1.shape), full(tower_W2.shape),
            full(tower_b2.shape),
        ],
        out_specs=pl.BlockSpec((bm, 2), lambda i: (i, 0)),
        out_shape=jax.ShapeDtypeStruct((m, 2), jnp.float32),
    )
    return fn(x, expert_W, expert_b, gate_W, tower_W1, tower_b1, tower_W2,
              tower_b2)


def kernel(target_user, item_sample, user_sample, embed, embed_ui, embed_pi,
           embed_u, W_hg, W_g1, W_g2, hyper_src, hyper_dst, ii_src, ii_dst,
           pi_src, pi_dst, ip_src, ip_dst, expert_W, expert_b, gate_W,
           tower_W1, tower_b1, tower_W2, tower_b2):
    N, U, I = N_NODES, U_NUM, I_NUM

    deg_ii, deg_pi, deg_ip, deg_hy = _sc_deg(
        [(ii_dst, N), (pi_dst, N), (ip_dst, U), (hyper_dst, N)])
    dinv_ii = 1.0 / jnp.clip(deg_ii, 1.0)
    dinv_pi = 1.0 / jnp.clip(deg_pi, 1.0)
    dinv_ip = 1.0 / jnp.clip(deg_ip, 1.0)
    dinv_hyper = 1.0 / jnp.clip(deg_hy, 1.0)

    t0 = _mm(embed, W_hg)
    sh_a, sh_b = _sc_pass(t0[:, :64], t0[:, 64:], hyper_src, hyper_dst, N)
    s_h = jnp.concatenate([sh_a, sh_b], axis=1)

    def gcn2(xh_pre, xh_dinv, relu_h, xg, src, dst, dinv, n):
        t1h = _mm(xh_pre, W_g1, scale=xh_dinv, relu_in=relu_h)
        t1g = _mm(xg, W_g1)
        s1h, s1g = _sc_pass(t1h, t1g, src, dst, n)
        t2h = _mm(s1h, W_g2, scale=dinv, relu_in=True)
        t2g = _mm(s1g, W_g2, scale=dinv, relu_in=True)
        s2h, s2g = _sc_pass(t2h, t2g, src, dst, n)
        return s2h * dinv, s2g * dinv

    init_item_h, init_item_g = gcn2(s_h, dinv_hyper, True, embed_ui,
                                    ii_src, ii_dst, dinv_ii, N)
    part_item_h, part_item_g = gcn2(s_h, dinv_hyper, True, embed_pi,
                                    pi_src, pi_dst, dinv_pi, N)
    init_part_h, init_part_g = gcn2(s_h[:U], dinv_hyper[:U], True, embed_u,
                                    ip_src, ip_dst, dinv_ip, U)

    init_item_embed = jnp.concatenate((init_item_h, init_item_g), axis=1)
    part_item_embed = jnp.concatenate((part_item_h, part_item_g), axis=1)
    init_part_embed = jnp.concatenate((init_part_h, init_part_g), axis=1)

    init_item_type = init_item_embed[:U]
    init_part_type = init_part_embed[:U]
    part_item_type = part_item_embed[:U]
    part_init_type = init_part_embed[:U]
    item_init_type = init_item_embed[U:U + I]
    item_part_type = part_item_embed[U:U + I]

    allp = jnp.mean(jnp.concatenate((part_item_type, part_init_type), axis=1),
                    axis=0, keepdims=True)
    tu = jnp.concatenate((init_item_type[target_user][:, None, :],
                          init_part_type[target_user][:, None, :]), axis=2)
    B, Si = item_sample.shape
    isf = item_sample.reshape(-1)
    item_sample_embed = jnp.concatenate(
        (item_init_type[isf].reshape(B, Si, -1),
         item_part_type[isf].reshape(B, Si, -1)), axis=2)
    Sp = user_sample.shape[1]
    usf = user_sample.reshape(-1)
    user_sample_embed = jnp.concatenate(
        (part_item_type[usf].reshape(B, Sp, -1),
         part_init_type[usf].reshape(B, Sp, -1)), axis=2)
    true_item = item_sample_embed[:, 0:1]
    users1 = jnp.tile(tu, (1, Si, 1))
    users2 = jnp.tile(tu, (1, Sp, 1))
    true_is = jnp.tile(true_item, (1, Sp, 1))
    allp_b = jnp.tile(allp[None], (B, Si, 1))
    u_isample_p = jnp.concatenate((users1, item_sample_embed, allp_b), axis=2)
    u_i_psample = jnp.concatenate((users2, true_is, user_sample_embed),
                                  axis=2)
    u_i_p = jnp.concatenate((u_isample_p, u_i_psample), axis=1)
    bs, ss, es = u_i_p.shape
    x = u_i_p.reshape(bs * ss, es)

    o = _mmoe(x, expert_W, expert_b, gate_W, tower_W1, tower_b1, tower_W2,
              tower_b2)
    output1 = o[:, 0].reshape(bs, ss)
    output2 = o[:, 1].reshape(bs, ss)
    loc = ss // 2
    task1_score = output1[:, :loc]
    task2_score = output2[:, loc:]

    def bpr(inp):
        return jnp.mean(-jax.nn.log_sigmoid(inp[:, 0:1] - inp[:, 1:]),
                        axis=-1)

    bprloss = 0.2 * bpr(task1_score[:, 0:5]) + bpr(task2_score[:, 0:5])
    truelabels = jnp.ones((bs, ss), jnp.float32).at[:, 1:loc].set(0.0)
    listloss = -jnp.sum(jax.nn.softmax(truelabels, axis=1)
                        * jnp.log(jax.nn.softmax(output1, axis=1)), axis=1)
    loss = bprloss + 0.3 * listloss + bpr(output2[:, :loc])
    return loss, task1_score, task2_score


# speedup vs baseline: 2.6950x; 1.0767x over previous
import functools
import math

import jax
import jax.numpy as jnp
from jax import lax
from jax.experimental import pallas as pl
from jax.experimental.pallas import tpu as pltpu
from jax.experimental.pallas import tpu_sc as plsc

N_NODES, U_NUM, I_NUM = 3000 + 7000, 3000, 7000
_NS, _NC, _L = 16, 2, 128


_G = 8


def _sc_pass_body(x_ref, src_ref, dst_ref, z_ref, out_ref,
                  outs, rows, src_v, dst_v, gsem, ssem, isem,
                  *, w, n_tab, n_groups):
    c = lax.axis_index("c")
    s = lax.axis_index("s")
    r_t = n_tab // _NS
    row0 = s * r_t
    n_slabs = r_t // _L
    pltpu.sync_copy(z_ref, rows.at[0])
    for v in range(n_slabs):
        pltpu.sync_copy(rows.at[0], outs.at[pl.ds(row0 + v * _L, _L)])
    plsc.subcore_barrier()

    pltpu.async_copy(src_ref.at[s, 0], src_v.at[0], isem)
    pltpu.async_copy(dst_ref.at[s, 0], dst_v.at[0], isem)

    @pl.loop(0, n_groups)
    def _grp(g):
        p = g % 2
        pltpu.make_async_copy(src_ref.at[s, g], src_v.at[p], isem).wait()
        pltpu.make_async_copy(dst_ref.at[s, g], dst_v.at[p], isem).wait()

        @pl.when(g + 1 < n_groups)
        def _prefetch():
            pltpu.async_copy(src_ref.at[s, g + 1], src_v.at[1 - p], isem)
            pltpu.async_copy(dst_ref.at[s, g + 1], dst_v.at[1 - p], isem)

        gh = [None] * _G
        sh = [None] * _G
        gh[0] = pltpu.async_copy(x_ref.at[c].at[src_v.at[p, 0]], rows.at[0],
                                 gsem)
        for b in range(_G):
            gh[b].wait()
            if b >= 1:
                sh[b - 1].wait()
            if b + 1 < _G:
                gh[b + 1] = pltpu.async_copy(
                    x_ref.at[c].at[src_v.at[p, b + 1]], rows.at[(b + 1) % 2],
                    gsem)
            sh[b] = pltpu.async_copy(rows.at[b % 2], outs.at[dst_v.at[p, b]],
                                     ssem, add=True)
        sh[_G - 1].wait()

    plsc.subcore_barrier()
    for v in range(n_slabs):
        pltpu.sync_copy(outs.at[pl.ds(row0 + v * _L, _L)], rows.at[0])
        pltpu.sync_copy(rows.at[0], out_ref.at[c, pl.ds(row0 + v * _L, _L)])


def _sc_pass(xa, xb, src, dst, n):
    w = xa.shape[1]
    n_tab = math.ceil((n + 8) / (_NS * _L)) * (_NS * _L)
    E = src.shape[0]
    epad = math.ceil(E / (_NS * _L * _G)) * (_NS * _L * _G)
    n_groups = epad // (_NS * _L * _G)
    x2 = jnp.pad(jnp.stack([xa, xb]), ((0, 0), (0, n_tab - n), (0, 0)))
    fill = jnp.full((epad - E,), n, jnp.int32)
    srcp = jnp.concatenate([src.astype(jnp.int32), fill]).reshape(
        _NS, n_groups, _G, _L)
    dstp = jnp.concatenate([dst.astype(jnp.int32), fill]).reshape(
        _NS, n_groups, _G, _L)
    zeros = jnp.zeros((_L, w), jnp.float32)
    fn = pl.kernel(
        functools.partial(_sc_pass_body, w=w, n_tab=n_tab, n_groups=n_groups),
        out_type=jax.ShapeDtypeStruct((_NC, n_tab, w), jnp.float32),
        mesh=plsc.VectorSubcoreMesh(core_axis_name="c", subcore_axis_name="s"),
        compiler_params=pltpu.CompilerParams(use_tc_tiling_on_sc=False),
        scratch_types=[
            pltpu.VMEM_SHARED((n_tab, w), jnp.float32),
            pltpu.VMEM((2, _L, w), jnp.float32),
            pltpu.VMEM((2, _G, _L), jnp.int32),
            pltpu.VMEM((2, _G, _L), jnp.int32),
            pltpu.SemaphoreType.DMA,
            pltpu.SemaphoreType.DMA,
            pltpu.SemaphoreType.DMA,
        ],
    )
    out = fn(x2, srcp, dstp, zeros)
    return out[0, :n], out[1, :n]


_GD = 32


def _sc_deg_body(dst_ref, one_ref, z_ref, out_ref,
                 degs, onev, stage, dst_v, ssem, isem, *, n_tab, n_groups):
    c = lax.axis_index("c")
    s = lax.axis_index("s")
    r_t = n_tab // _NS
    row0 = s * r_t
    n_slabs = r_t // _L
    pltpu.sync_copy(one_ref, onev)
    pltpu.sync_copy(z_ref, stage)
    for v in range(n_slabs):
        pltpu.sync_copy(stage, degs.at[pl.ds(row0 + v * _L, _L)])
    plsc.subcore_barrier()
    pltpu.async_copy(dst_ref.at[c, s, 0], dst_v.at[0], isem)

    @pl.loop(0, n_groups)
    def _grp(g):
        p = g % 2
        pltpu.make_async_copy(dst_ref.at[c, s, g], dst_v.at[p], isem).wait()

        @pl.when(g + 1 < n_groups)
        def _prefetch():
            pltpu.async_copy(dst_ref.at[c, s, g + 1], dst_v.at[1 - p], isem)

        sh = []
        for b in range(_GD):
            sh.append(pltpu.async_copy(onev, degs.at[dst_v.at[p, b]], ssem,
                                       add=True))
        for h in sh:
            h.wait()

    plsc.subcore_barrier()
    for v in range(n_slabs):
        pltpu.sync_copy(degs.at[pl.ds(row0 + v * _L, _L)], stage)
        pltpu.sync_copy(stage, out_ref.at[c, pl.ds(row0 + v * _L, _L)])


def _sc_deg(dsts_ns):
    n_big = max(n for _, n in dsts_ns)
    base = math.ceil((n_big + 8) / (_NS * _L)) * (_NS * _L)
    offs = []
    o = 0
    for _, n in dsts_ns:
        offs.append(o)
        o += base
    cap = _NS * _L * _GD
    etot = sum(d.shape[0] for d, _ in dsts_ns)
    n_groups = math.ceil(etot / (_NC * cap))
    epad = n_groups * _NC * cap
    n_tab = o
    junk = n_tab - 1
    parts = [d.astype(jnp.int32) + off for (d, _), off in zip(dsts_ns, offs)]
    parts.append(jnp.full((epad - etot,), junk, jnp.int32))
    dstp = jnp.concatenate(parts).reshape(_NC, _NS, n_groups, _GD, _L)
    ones = jnp.ones((_L, 16), jnp.float32)
    zeros = jnp.zeros((_L, 16), jnp.float32)
    fn = pl.kernel(
        functools.partial(_sc_deg_body, n_tab=n_tab, n_groups=n_groups),
        out_type=jax.ShapeDtypeStruct((_NC, n_tab, 16), jnp.float32),
        mesh=plsc.VectorSubcoreMesh(core_axis_name="c", subcore_axis_name="s"),
        compiler_params=pltpu.CompilerParams(use_tc_tiling_on_sc=False),
        scratch_types=[
            pltpu.VMEM_SHARED((n_tab, 16), jnp.float32),
            pltpu.VMEM((_L, 16), jnp.float32),
            pltpu.VMEM((_L, 16), jnp.float32),
            pltpu.VMEM((2, _GD, _L), jnp.int32),
            pltpu.SemaphoreType.DMA,
            pltpu.SemaphoreType.DMA,
        ],
    )
    out = fn(dstp, ones, zeros)
    deg = out[0, :, 0:1] + out[1, :, 0:1]
    return [deg[off:off + n] for (_, n), off in zip(dsts_ns, offs)]


def _mm_body(x_ref, w_ref, o_ref, *, relu_in):
    x = x_ref[...]
    if relu_in:
        x = jnp.maximum(x, 0.0)
    o_ref[...] = jnp.dot(x, w_ref[...], preferred_element_type=jnp.float32)


def _mm_scale_body(x_ref, s_ref, w_ref, o_ref, *, relu_in):
    x = x_ref[...]
    if relu_in:
        x = jnp.maximum(x, 0.0)
    x = x * s_ref[...]
    o_ref[...] = jnp.dot(x, w_ref[...], preferred_element_type=jnp.float32)


def _mm(x, w, scale=None, relu_in=False, bm=512):
    n, d = x.shape
    npad = math.ceil(n / bm) * bm
    if npad != n:
        x = jnp.pad(x, ((0, npad - n), (0, 0)))
        if scale is not None:
            scale = jnp.pad(scale, ((0, npad - n), (0, 0)))
    grid = (npad // bm,)
    out_shape = jax.ShapeDtypeStruct((npad, w.shape[1]), jnp.float32)
    if scale is None:
        fn = pl.pallas_call(
            functools.partial(_mm_body, relu_in=relu_in),
            grid=grid,
            in_specs=[
                pl.BlockSpec((bm, d), lambda i: (i, 0)),
                pl.BlockSpec((d, w.shape[1]), lambda i: (0, 0)),
            ],
            out_specs=pl.BlockSpec((bm, w.shape[1]), lambda i: (i, 0)),
            out_shape=out_shape,
        )
        y = fn(x, w)
    else:
        fn = pl.pallas_call(
            functools.partial(_mm_scale_body, relu_in=relu_in),
            grid=grid,
            in_specs=[
                pl.BlockSpec((bm, d), lambda i: (i, 0)),
                pl.BlockSpec((bm, 1), lambda i: (i, 0)),
                pl.BlockSpec((d, w.shape[1]), lambda i: (0, 0)),
            ],
            out_specs=pl.BlockSpec((bm, w.shape[1]), lambda i: (i, 0)),
            out_shape=out_shape,
        )
        y = fn(x, scale, w)
    return y[:n]


def _mmoe_body(u_ref, it_ref, us_ref, ap_ref, ew_ref, eb_ref, gw_ref,
               tw1_ref, tb1_ref, tw2_ref, tb2_ref, o_ref, *, bb, si):
    u = u_ref[...]
    it = it_ref[...]
    us = us_ref[...]
    ap = ap_ref[...]
    dot = lambda a, b: jnp.dot(a, b, preferred_element_type=jnp.float32)
    eo1, eo2 = [], []
    for k in range(6):
        a = dot(u, ew_ref[k, 0:512])
        b3 = dot(it, ew_ref[k, 512:1024]).reshape(bb, si, 256)
        c3 = dot(us, ew_ref[k, 1024:1536]).reshape(bb, si, 256)
        apk = dot(ap, ew_ref[k, 1024:1536])
        base = a[:, None, :] + eb_ref[k][None, None, :]
        eo1.append(jnp.maximum(base + b3 + apk[None], 0.0))
        eo2.append(jnp.maximum(base + b3[:, 0:1, :] + c3, 0.0))
    outs = []
    for t in range(2):
        gu = dot(u, gw_ref[t, 0:512])
        gb3 = dot(it, gw_ref[t, 512:1024]).reshape(bb, si, 6)
        gc3 = dot(us, gw_ref[t, 1024:1536]).reshape(bb, si, 6)
        gap = dot(ap, gw_ref[t, 1024:1536])
        l1 = gu[:, None, :] + gb3 + gap[None]
        l2 = gu[:, None, :] + gb3[:, 0:1, :] + gc3
        row_out = []
        for l, eo in ((l1, eo1), (l2, eo2)):
            l = l - jnp.max(l, axis=-1, keepdims=True)
            eg = jnp.exp(l)
            g = eg / jnp.sum(eg, axis=-1, keepdims=True)
            mixed = g[..., 0:1] * eo[0]
            for k in range(1, 6):
                mixed = mixed + g[..., k:k + 1] * eo[k]
            t1 = dot(mixed.reshape(bb * si, 256), tw1_ref[t])
            t1 = jnp.maximum(t1 + tb1_ref[t][None, :], 0.0)
            t2 = dot(t1, tw2_ref[t]) + tb2_ref[t][0]
            row_out.append(t2.reshape(bb, si))
        outs.append(jnp.concatenate(row_out, axis=1))
    o_ref[...] = jnp.stack(outs, axis=-1)


def _mmoe(u, it, us, ap, expert_W, expert_b, gate_W, tower_W1, tower_b1,
          tower_W2, tower_b2, bb=64):
    B = u.shape[0]
    si = it.shape[0] // B
    grid = (B // bb,)
    full = lambda shp: pl.BlockSpec(shp, lambda i: tuple(0 for _ in shp))
    fn = pl.pallas_call(
        functools.partial(_mmoe_body, bb=bb, si=si),
        grid=grid,
        in_specs=[
            pl.BlockSpec((bb, 512), lambda i: (i, 0)),
            pl.BlockSpec((bb * si, 512), lambda i: (i, 0)),
            pl.BlockSpec((bb * si, 512), lambda i: (i, 0)),
            full(ap.shape),
            full(expert_W.shape), full(expert_b.shape), full(gate_W.shape),
            full(tower_W1.shape), full(tower_b1.shape), full(tower_W2.shape),
            full(tower_b2.shape),
        ],
        out_specs=pl.BlockSpec((bb, 2 * si, 2), lambda i: (i, 0, 0)),
        out_shape=jax.ShapeDtypeStruct((B, 2 * si, 2), jnp.float32),
    )
    return fn(u, it, us, ap, expert_W, expert_b, gate_W, tower_W1, tower_b1,
              tower_W2, tower_b2)


def kernel(target_user, item_sample, user_sample, embed, embed_ui, embed_pi,
           embed_u, W_hg, W_g1, W_g2, hyper_src, hyper_dst, ii_src, ii_dst,
           pi_src, pi_dst, ip_src, ip_dst, expert_W, expert_b, gate_W,
           tower_W1, tower_b1, tower_W2, tower_b2):
    N, U, I = N_NODES, U_NUM, I_NUM

    deg_ii, deg_pi, deg_ip, deg_hy = _sc_deg(
        [(ii_dst, N), (pi_dst, N), (ip_dst, U), (hyper_dst, N)])
    dinv_ii = 1.0 / jnp.clip(deg_ii, 1.0)
    dinv_pi = 1.0 / jnp.clip(deg_pi, 1.0)
    dinv_ip = 1.0 / jnp.clip(deg_ip, 1.0)
    dinv_hyper = 1.0 / jnp.clip(deg_hy, 1.0)

    t0 = _mm(embed, W_hg)
    sh_a, sh_b = _sc_pass(t0[:, :64], t0[:, 64:], hyper_src, hyper_dst, N)
    s_h = jnp.concatenate([sh_a, sh_b], axis=1)

    def gcn2(xh_pre, xh_dinv, relu_h, xg, src, dst, dinv, n):
        t1h = _mm(xh_pre, W_g1, scale=xh_dinv, relu_in=relu_h)
        t1g = _mm(xg, W_g1)
        s1h, s1g = _sc_pass(t1h, t1g, src, dst, n)
        t2h = _mm(s1h, W_g2, scale=dinv, relu_in=True)
        t2g = _mm(s1g, W_g2, scale=dinv, relu_in=True)
        s2h, s2g = _sc_pass(t2h, t2g, src, dst, n)
        return s2h * dinv, s2g * dinv

    init_item_h, init_item_g = gcn2(s_h, dinv_hyper, True, embed_ui,
                                    ii_src, ii_dst, dinv_ii, N)
    part_item_h, part_item_g = gcn2(s_h, dinv_hyper, True, embed_pi,
                                    pi_src, pi_dst, dinv_pi, N)
    init_part_h, init_part_g = gcn2(s_h[:U], dinv_hyper[:U], True, embed_u,
                                    ip_src, ip_dst, dinv_ip, U)

    init_item_embed = jnp.concatenate((init_item_h, init_item_g), axis=1)
    part_item_embed = jnp.concatenate((part_item_h, part_item_g), axis=1)
    init_part_embed = jnp.concatenate((init_part_h, init_part_g), axis=1)

    init_item_type = init_item_embed[:U]
    init_part_type = init_part_embed[:U]
    part_item_type = part_item_embed[:U]
    part_init_type = init_part_embed[:U]
    item_init_type = init_item_embed[U:U + I]
    item_part_type = part_item_embed[U:U + I]

    allp = jnp.mean(jnp.concatenate((part_item_type, part_init_type), axis=1),
                    axis=0, keepdims=True)
    B, Si = item_sample.shape
    isf = item_sample.reshape(-1)
    usf = user_sample.reshape(-1)
    u = jnp.concatenate((init_item_type[target_user],
                         init_part_type[target_user]), axis=1)
    it = jnp.concatenate((item_init_type[isf], item_part_type[isf]), axis=1)
    us = jnp.concatenate((part_item_type[usf], part_init_type[usf]), axis=1)

    o = _mmoe(u, it, us, allp, expert_W, expert_b, gate_W, tower_W1,
              tower_b1, tower_W2, tower_b2)
    bs, ss = B, 2 * Si
    output1 = o[..., 0]
    output2 = o[..., 1]
    loc = ss // 2
    task1_score = output1[:, :loc]
    task2_score = output2[:, loc:]

    def bpr(inp):
        return jnp.mean(-jax.nn.log_sigmoid(inp[:, 0:1] - inp[:, 1:]),
                        axis=-1)

    bprloss = 0.2 * bpr(task1_score[:, 0:5]) + bpr(task2_score[:, 0:5])
    truelabels = jnp.ones((bs, ss), jnp.float32).at[:, 1:loc].set(0.0)
    listloss = -jnp.sum(jax.nn.softmax(truelabels, axis=1)
                        * jnp.log(jax.nn.softmax(output1, axis=1)), axis=1)
    loss = bprloss + 0.3 * listloss + bpr(output2[:, :loc])
    return loss, task1_score, task2_score
